# baseline (speedup 1.0000x reference)
import functools

import jax
import jax.numpy as jnp
from jax import lax
from jax.experimental import pallas as pl
from jax.experimental.pallas import tpu as pltpu
from jax.experimental.pallas import tpu_sc as plsc

VOCAB = 1000000
EMBED = 16
BATCH = 4096

_NC = 2
_NS = 16
_NW = _NC * _NS
_BPW = BATCH // _NW


def _sc_gather_kernel():
    mesh = plsc.VectorSubcoreMesh(core_axis_name="c", subcore_axis_name="s")

    @functools.partial(
        pl.kernel,
        mesh=mesh,
        compiler_params=pltpu.CompilerParams(use_tc_tiling_on_sc=False),
        out_type=(
            jax.ShapeDtypeStruct((BATCH, EMBED), jnp.float32),
            jax.ShapeDtypeStruct((BATCH, EMBED), jnp.float32),
        ),
        scratch_types=[
            pltpu.VMEM((_BPW,), jnp.int32),
            pltpu.VMEM((_BPW, EMBED), jnp.float32),
            pltpu.VMEM((_BPW,), jnp.int32),
            pltpu.VMEM((_BPW, EMBED), jnp.float32),
            pltpu.SemaphoreType.DMA,
            pltpu.SemaphoreType.DMA,
        ],
    )
    def gather(cw_hbm, xw_hbm, v_hbm, u_hbm, outv_hbm, outu_hbm,
               idx_c, rows_c, idx_x, rows_x, sem_c, sem_x):
        wid = lax.axis_index("s") * _NC + lax.axis_index("c")
        base = wid * _BPW
        pltpu.sync_copy(cw_hbm.at[pl.ds(base, _BPW)], idx_c)
        pltpu.sync_copy(xw_hbm.at[pl.ds(base, _BPW)], idx_x)
        cp_c = pltpu.async_copy(v_hbm.at[idx_c], rows_c, sem_c)
        cp_x = pltpu.async_copy(u_hbm.at[idx_x], rows_x, sem_x)
        cp_c.wait()
        pltpu.sync_copy(rows_c, outv_hbm.at[pl.ds(base, _BPW)])
        cp_x.wait()
        pltpu.sync_copy(rows_x, outu_hbm.at[pl.ds(base, _BPW)])

    return gather


_ROW_TILE = 256


def _score_softmax_body(c_ref, x_ref, o_ref):
    scores = lax.dot_general(
        c_ref[...], x_ref[...],
        dimension_numbers=(((1,), (1,)), ((), ())),
        preferred_element_type=jnp.float32,
    )
    m = jnp.max(scores, axis=1, keepdims=True)
    e = jnp.exp(scores - m)
    s = jnp.sum(e, axis=1, keepdims=True)
    o_ref[...] = (scores - m) - jnp.log(s)


def kernel(center_words, context_words, embedding_v, embedding_u):
    center_embed, context_embed = _sc_gather_kernel()(
        center_words.astype(jnp.int32), context_words.astype(jnp.int32),
        embedding_v, embedding_u)

    log_probs = pl.pallas_call(
        _score_softmax_body,
        grid=(BATCH // _ROW_TILE,),
        in_specs=[
            pl.BlockSpec((_ROW_TILE, EMBED), lambda i: (i, 0)),
            pl.BlockSpec((BATCH, EMBED), lambda i: (0, 0)),
        ],
        out_specs=pl.BlockSpec((_ROW_TILE, BATCH), lambda i: (i, 0)),
        out_shape=jax.ShapeDtypeStruct((BATCH, BATCH), jnp.float32),
    )(center_embed, context_embed)
    return log_probs
